# baseline (speedup 1.0000x reference)
import jax
import jax.numpy as jnp
from jax.experimental import pallas as pl
from jax.experimental.pallas import tpu as pltpu

_C = 256
_K = 32
_D = 16


def _dma_body(in_ref, out_ref, buf, isem, osem):
    total = in_ref.shape[0]
    nch = total // _C

    def fire_in(j):
        return pltpu.make_async_copy(
            in_ref.at[pl.ds(j * _C, _C)], buf.at[j % _K], isem
        )

    in_h = {}
    out_h = {}
    for j in range(min(_D, nch)):
        in_h[j] = fire_in(j)
        in_h[j].start()
    waited_out = 0
    for j in range(nch):
        nxt = j + _D
        if nxt < nch:
            if nxt >= _K:
                out_h[nxt - _K].wait()
                waited_out = nxt - _K + 1
            in_h[nxt] = fire_in(nxt)
            in_h[nxt].start()
        in_h[j].wait()
        out_h[j] = pltpu.make_async_copy(
            buf.at[j % _K], out_ref.at[pl.ds(j * _C, _C)], osem
        )
        out_h[j].start()
    for j in range(waited_out, nch):
        out_h[j].wait()


def kernel(inputs, mask):
    total, H = inputs.shape
    B, L = mask.shape
    out = pl.pallas_call(
        _dma_body,
        in_specs=[pl.BlockSpec(memory_space=pl.ANY)],
        out_specs=pl.BlockSpec(memory_space=pl.ANY),
        out_shape=jax.ShapeDtypeStruct((total, H), inputs.dtype),
        scratch_shapes=[
            pltpu.VMEM((_K, _C, H), inputs.dtype),
            pltpu.SemaphoreType.DMA,
            pltpu.SemaphoreType.DMA,
        ],
    )(inputs)
    return out.reshape(B, L, H), mask


# speedup vs baseline: 1.0067x; 1.0067x over previous
import jax
import jax.numpy as jnp
from jax.experimental import pallas as pl
from jax.experimental.pallas import tpu as pltpu

_C = 1024
_K = 8
_D = 4


def _dma_body(in_ref, out_ref, buf, isem, osem):
    total = in_ref.shape[0]
    nch = total // _C

    def fire_in(j):
        return pltpu.make_async_copy(
            in_ref.at[pl.ds(j * _C, _C)], buf.at[j % _K], isem
        )

    in_h = {}
    out_h = {}
    for j in range(min(_D, nch)):
        in_h[j] = fire_in(j)
        in_h[j].start()
    waited_out = 0
    for j in range(nch):
        nxt = j + _D
        if nxt < nch:
            if nxt >= _K:
                out_h[nxt - _K].wait()
                waited_out = nxt - _K + 1
            in_h[nxt] = fire_in(nxt)
            in_h[nxt].start()
        in_h[j].wait()
        out_h[j] = pltpu.make_async_copy(
            buf.at[j % _K], out_ref.at[pl.ds(j * _C, _C)], osem
        )
        out_h[j].start()
    for j in range(waited_out, nch):
        out_h[j].wait()


def kernel(inputs, mask):
    total, H = inputs.shape
    B, L = mask.shape
    out = pl.pallas_call(
        _dma_body,
        in_specs=[pl.BlockSpec(memory_space=pl.ANY)],
        out_specs=pl.BlockSpec(memory_space=pl.ANY),
        out_shape=jax.ShapeDtypeStruct((total, H), inputs.dtype),
        scratch_shapes=[
            pltpu.VMEM((_K, _C, H), inputs.dtype),
            pltpu.SemaphoreType.DMA,
            pltpu.SemaphoreType.DMA,
        ],
    )(inputs)
    return out.reshape(B, L, H), mask
